# baseline (speedup 1.0000x reference)
import functools

import jax
import jax.numpy as jnp
from jax import lax
from jax.experimental import pallas as pl
from jax.experimental.pallas import tpu as pltpu
from jax.experimental.pallas import tpu_sc as plsc

_VOCAB = 100000
_E = 8
_D_SUB = 128
_D_TRAIN = 512
_B, _L = 1024, 50
_N = _B * _L
_NC = 2
_NS = 16
_NW = _NC * _NS
_TPW = _N // _NW
_CT = 80
_NCH = _TPW // _CT


def _body(x_hbm, t_hbm, w_hbm, out_hbm, xv, idxb,
          buf_a0, buf_a1, buf_a2, buf_a3, buf_b0, buf_b1,
          sga0, sga1, sga2, sga3, swa0, swa1, swa2, swa3,
          sgb0, sgb1, swb0, swb1):
    cid = lax.axis_index("c")
    sid = lax.axis_index("s")
    wid = sid * _NC + cid
    base = wid * _TPW
    buf_a = (buf_a0, buf_a1, buf_a2, buf_a3)
    buf_b = (buf_b0, buf_b1)
    sga = (sga0, sga1, sga2, sga3)
    swa = (swa0, swa1, swa2, swa3)
    sgb = (sgb0, sgb1)
    swb = (swb0, swb1)

    pltpu.sync_copy(x_hbm.at[pl.ds(base, _TPW)], xv)

    def build(c, carry):
        for k in range(_CT // 16):
            v = xv[pl.ds(c * _CT + k * 16, 16)]
            idxb[c, pl.ds(k * 16, 16)] = v
        return carry

    lax.fori_loop(0, _NCH, build, 0)

    def b_gather(c, h):
        return pltpu.async_copy(w_hbm.at[idxb.at[c]], buf_b[h], sgb[h])

    def b_write(c, h):
        return pltpu.make_async_copy(
            buf_b[h],
            out_hbm.at[pl.ds(base + c * _CT, _CT),
                       pl.ds(_E * _D_SUB, _D_TRAIN)],
            swb[h])

    def a_gather(c, e):
        return pltpu.async_copy(
            t_hbm.at[e].at[idxb.at[c]], buf_a[e % 4], sga[e % 4])

    def a_gather_wait(c, e):
        pltpu.make_async_copy(
            t_hbm.at[e].at[idxb.at[c]], buf_a[e % 4], sga[e % 4]).wait()

    def a_write(c, e):
        return pltpu.make_async_copy(
            buf_a[e % 4],
            out_hbm.at[pl.ds(base + c * _CT, _CT),
                       pl.ds(e * _D_SUB, _D_SUB)],
            swa[e % 4])

    b_gather(0, 0)
    b_gather(1, 1)

    def chunk(c, c2, h, first):
        pltpu.make_async_copy(w_hbm.at[idxb.at[c]], buf_b[h], sgb[h]).wait()
        b_write(c, h).start()
        for e in range(_E):
            if e >= 4:
                a_write(c, e - 4).wait()
            elif not first:
                a_write(c - 1, 4 + e).wait()
            a_gather(c, e)
            if e >= 1:
                a_gather_wait(c, e - 1)
                a_write(c, e - 1).start()
            elif not first:
                a_gather_wait(c - 1, 7)
                a_write(c - 1, 7).start()
        b_write(c, h).wait()
        if isinstance(c2, int):
            if c2 < _NCH // 2 - 1:
                b_gather(c + 2, h)
        else:
            @pl.when(c2 < _NCH // 2 - 1)
            def _():
                b_gather(c + 2, h)

    chunk(0, 0, 0, True)
    chunk(1, 0, 1, False)

    def step(c2, carry):
        chunk(2 * c2, c2, 0, False)
        chunk(2 * c2 + 1, c2, 1, False)
        return carry

    lax.fori_loop(1, _NCH // 2, step, 0)

    a_gather_wait(_NCH - 1, 7)
    a_write(_NCH - 1, 7).start()
    for e in (4, 5, 6, 7):
        a_write(_NCH - 1, e).wait()


@jax.jit
def _lookup(x_flat, tables, W_train):
    f = functools.partial(
        pl.kernel,
        mesh=plsc.VectorSubcoreMesh(core_axis_name="c", subcore_axis_name="s"),
        out_type=jax.ShapeDtypeStruct((_N, _E * _D_SUB + _D_TRAIN),
                                      jnp.float32),
        scratch_types=[
            pltpu.VMEM((_TPW,), jnp.int32),
            pltpu.VMEM((_NCH, _CT), jnp.int32),
            pltpu.VMEM((_CT, _D_SUB), jnp.float32),
            pltpu.VMEM((_CT, _D_SUB), jnp.float32),
            pltpu.VMEM((_CT, _D_SUB), jnp.float32),
            pltpu.VMEM((_CT, _D_SUB), jnp.float32),
            pltpu.VMEM((_CT, _D_TRAIN), jnp.float32),
            pltpu.VMEM((_CT, _D_TRAIN), jnp.float32),
        ] + [pltpu.SemaphoreType.DMA] * 12,
    )(_body)
    return f(x_flat, tables, W_train)


def kernel(x, tables, W_train):
    x_lm = x.T.reshape(_N).astype(jnp.int32)
    out = _lookup(x_lm, tables, W_train)
    out = out.reshape(_L, _B, _E * _D_SUB + _D_TRAIN)
    return out.transpose(1, 0, 2)
